# baseline (speedup 1.0000x reference)
import jax
import jax.numpy as jnp
from jax import lax
from jax.experimental import pallas as pl
from jax.experimental.pallas import tpu as pltpu
from jax.experimental.pallas import tpu_sc as plsc

NUM_NODES = 100000
NUM_TYPES_ROWS = 64
EMBED_DIM = 64
LANES = 16
CHUNK = 512
NGRP = CHUNK // LANES
NFULL = NUM_NODES // CHUNK
TAIL = NUM_NODES - NFULL * CHUNK
TAIL_ALIGNED = 128
TAIL_REST = TAIL - TAIL_ALIGNED
NUM_WORKERS = 32
KSTEPS = -(-NFULL // NUM_WORKERS)
TAIL_WID = 3

_mesh = plsc.VectorSubcoreMesh(core_axis_name="c", subcore_axis_name="s")


@pl.kernel(
    mesh=_mesh,
    compiler_params=pltpu.CompilerParams(needs_layout_passes=False),
    out_type=(jax.ShapeDtypeStruct((EMBED_DIM, NUM_NODES), jnp.float32),
              jax.ShapeDtypeStruct((EMBED_DIM, TAIL_REST), jnp.float32)),
    scratch_types=[
        pltpu.VMEM((NUM_TYPES_ROWS * EMBED_DIM,), jnp.float32),
        pltpu.VMEM((CHUNK,), jnp.int32),
        pltpu.VMEM((CHUNK,), jnp.int32),
        pltpu.VMEM((EMBED_DIM, CHUNK), jnp.float32),
        pltpu.VMEM((EMBED_DIM, CHUNK), jnp.float32),
        pltpu.VMEM((EMBED_DIM, TAIL_REST), jnp.float32),
        pltpu.SemaphoreType.DMA,
        pltpu.SemaphoreType.DMA,
        pltpu.SemaphoreType.DMA,
        pltpu.SemaphoreType.DMA,
    ],
)
def _gather_kernel(idx_hbm, table_hbm, out_hbm, tail_hbm, tab_v, idx0, idx1,
                   obuf0, obuf1, ptail_v, semi0, semi1, semw0, semw1):
    wid = lax.axis_index("s") * 2 + lax.axis_index("c")
    idx_b = (idx0, idx1)
    obuf_b = (obuf0, obuf1)
    semi_b = (semi0, semi1)
    semw_b = (semw0, semw1)

    pltpu.sync_copy(table_hbm, tab_v)

    def stage_idx(k):
        chunk = k * NUM_WORKERS + wid

        @pl.when(chunk < NFULL)
        def _():
            b = k % 2
            pltpu.async_copy(idx_hbm.at[pl.ds(chunk * CHUNK, CHUNK)],
                             idx_b[b], semi_b[b])

    def compute(k):
        chunk = k * NUM_WORKERS + wid

        @pl.when(chunk < NFULL)
        def _():
            b = k % 2
            pltpu.make_async_copy(idx_hbm.at[pl.ds(chunk * CHUNK, CHUNK)],
                                  idx_b[b], semi_b[b]).wait()
            if k >= 2:
                prev = (k - 2) * NUM_WORKERS + wid
                pltpu.make_async_copy(
                    obuf_b[b],
                    out_hbm.at[:, pl.ds(prev * CHUNK, CHUNK)],
                    semw_b[b]).wait()

            def grp(g, carry):
                iv = idx_b[b][pl.ds(g * LANES, LANES)] * EMBED_DIM
                for c in range(EMBED_DIM):
                    vals = plsc.load_gather(tab_v, [iv + c])
                    obuf_b[b][c, pl.ds(g * LANES, LANES)] = vals
                return carry

            lax.fori_loop(0, NGRP, grp, 0)
            pltpu.async_copy(obuf_b[b],
                             out_hbm.at[:, pl.ds(chunk * CHUNK, CHUNK)],
                             semw_b[b])

    def drain(k):
        chunk = k * NUM_WORKERS + wid

        @pl.when(chunk < NFULL)
        def _():
            b = k % 2
            pltpu.make_async_copy(
                obuf_b[b],
                out_hbm.at[:, pl.ds(chunk * CHUNK, CHUNK)],
                semw_b[b]).wait()

    stage_idx(0)
    for k in range(KSTEPS):
        if k + 1 < KSTEPS:
            stage_idx(k + 1)
        compute(k)
    for k in range(max(KSTEPS - 2, 0), KSTEPS):
        drain(k)

    @pl.when(wid == TAIL_WID)
    def _():
        base = NFULL * CHUNK
        pltpu.sync_copy(idx_hbm.at[pl.ds(base, TAIL)],
                        idx0.at[pl.ds(0, TAIL)])

        def grp(g, carry):
            iv = idx0[pl.ds(g * LANES, LANES)] * EMBED_DIM
            for c in range(EMBED_DIM):
                vals = plsc.load_gather(tab_v, [iv + c])
                obuf0[c, pl.ds(g * LANES, LANES)] = vals
            return carry

        lax.fori_loop(0, TAIL_ALIGNED // LANES, grp, 0)
        pltpu.sync_copy(obuf0.at[:, pl.ds(0, TAIL_ALIGNED)],
                        out_hbm.at[:, pl.ds(base, TAIL_ALIGNED)])

        def grp_rest(g, carry):
            iv = idx0[pl.ds(TAIL_ALIGNED + g * LANES, LANES)] * EMBED_DIM
            for c in range(EMBED_DIM):
                vals = plsc.load_gather(tab_v, [iv + c])
                ptail_v[c, pl.ds(g * LANES, LANES)] = vals
            return carry

        lax.fori_loop(0, TAIL_REST // LANES, grp_rest, 0)
        pltpu.sync_copy(ptail_v, tail_hbm)


def kernel(type_indices, type_embedding):
    main_t, tail_t = _gather_kernel(type_indices.astype(jnp.int32),
                                    type_embedding.reshape(-1))
    out_t = lax.dynamic_update_slice(main_t, tail_t,
                                     (0, NFULL * CHUNK + TAIL_ALIGNED))
    return out_t.T


# speedup vs baseline: 1.2810x; 1.2810x over previous
import jax
import jax.numpy as jnp
from jax import lax
from jax.experimental import pallas as pl
from jax.experimental.pallas import tpu as pltpu
from jax.experimental.pallas import tpu_sc as plsc

NUM_NODES = 100000
NUM_TYPES_ROWS = 64
EMBED_DIM = 64
LANES = 16
CHUNK = 512
NGRP = CHUNK // LANES
NFULL = NUM_NODES // CHUNK
TAIL = NUM_NODES - NFULL * CHUNK
TAIL_ALIGNED = 128
TAIL_REST = TAIL - TAIL_ALIGNED
NUM_WORKERS = 32
KSTEPS = -(-NFULL // NUM_WORKERS)
TAIL_WID = 3

_mesh = plsc.VectorSubcoreMesh(core_axis_name="c", subcore_axis_name="s")


@pl.kernel(
    mesh=_mesh,
    compiler_params=pltpu.CompilerParams(needs_layout_passes=False),
    out_type=(jax.ShapeDtypeStruct((EMBED_DIM, NUM_NODES), jnp.float32),
              jax.ShapeDtypeStruct((EMBED_DIM, TAIL_REST), jnp.float32)),
    scratch_types=[
        pltpu.VMEM((NUM_TYPES_ROWS * EMBED_DIM,), jnp.float32),
        pltpu.VMEM((CHUNK,), jnp.int32),
        pltpu.VMEM((CHUNK,), jnp.int32),
        pltpu.VMEM((EMBED_DIM, CHUNK), jnp.float32),
        pltpu.VMEM((EMBED_DIM, CHUNK), jnp.float32),
        pltpu.VMEM((EMBED_DIM, TAIL_REST), jnp.float32),
        pltpu.SemaphoreType.DMA,
        pltpu.SemaphoreType.DMA,
        pltpu.SemaphoreType.DMA,
        pltpu.SemaphoreType.DMA,
    ],
)
def _gather_kernel(idx_hbm, table_hbm, out_hbm, tail_hbm, tab_v, idx0, idx1,
                   obuf0, obuf1, ptail_v, semi0, semi1, semw0, semw1):
    wid = lax.axis_index("s") * 2 + lax.axis_index("c")
    idx_b = (idx0, idx1)
    obuf_b = (obuf0, obuf1)
    semi_b = (semi0, semi1)
    semw_b = (semw0, semw1)

    pltpu.sync_copy(table_hbm, tab_v)

    def stage_idx(k):
        chunk = k * NUM_WORKERS + wid

        @pl.when(chunk < NFULL)
        def _():
            b = k % 2
            pltpu.async_copy(idx_hbm.at[pl.ds(chunk * CHUNK, CHUNK)],
                             idx_b[b], semi_b[b])

    def compute(k):
        chunk = k * NUM_WORKERS + wid

        @pl.when(chunk < NFULL)
        def _():
            b = k % 2
            pltpu.make_async_copy(idx_hbm.at[pl.ds(chunk * CHUNK, CHUNK)],
                                  idx_b[b], semi_b[b]).wait()
            if k >= 2:
                prev = (k - 2) * NUM_WORKERS + wid
                pltpu.make_async_copy(
                    obuf_b[b],
                    out_hbm.at[:, pl.ds(prev * CHUNK, CHUNK)],
                    semw_b[b]).wait()

            @plsc.parallel_loop(0, NGRP, unroll=4)
            def grp(g):
                iv = idx_b[b][pl.ds(g * LANES, LANES)] * EMBED_DIM
                for c in range(EMBED_DIM):
                    vals = plsc.load_gather(tab_v, [iv + c])
                    obuf_b[b][c, pl.ds(g * LANES, LANES)] = vals
            pltpu.async_copy(obuf_b[b],
                             out_hbm.at[:, pl.ds(chunk * CHUNK, CHUNK)],
                             semw_b[b])

    def drain(k):
        chunk = k * NUM_WORKERS + wid

        @pl.when(chunk < NFULL)
        def _():
            b = k % 2
            pltpu.make_async_copy(
                obuf_b[b],
                out_hbm.at[:, pl.ds(chunk * CHUNK, CHUNK)],
                semw_b[b]).wait()

    stage_idx(0)
    for k in range(KSTEPS):
        if k + 1 < KSTEPS:
            stage_idx(k + 1)
        compute(k)
    for k in range(max(KSTEPS - 2, 0), KSTEPS):
        drain(k)

    @pl.when(wid == TAIL_WID)
    def _():
        base = NFULL * CHUNK
        pltpu.sync_copy(idx_hbm.at[pl.ds(base, TAIL)],
                        idx0.at[pl.ds(0, TAIL)])

        @plsc.parallel_loop(0, TAIL_ALIGNED // LANES, unroll=2)
        def grp(g):
            iv = idx0[pl.ds(g * LANES, LANES)] * EMBED_DIM
            for c in range(EMBED_DIM):
                vals = plsc.load_gather(tab_v, [iv + c])
                obuf0[c, pl.ds(g * LANES, LANES)] = vals
        pltpu.sync_copy(obuf0.at[:, pl.ds(0, TAIL_ALIGNED)],
                        out_hbm.at[:, pl.ds(base, TAIL_ALIGNED)])

        @plsc.parallel_loop(0, TAIL_REST // LANES, unroll=1)
        def grp_rest(g):
            iv = idx0[pl.ds(TAIL_ALIGNED + g * LANES, LANES)] * EMBED_DIM
            for c in range(EMBED_DIM):
                vals = plsc.load_gather(tab_v, [iv + c])
                ptail_v[c, pl.ds(g * LANES, LANES)] = vals
        pltpu.sync_copy(ptail_v, tail_hbm)


def kernel(type_indices, type_embedding):
    main_t, tail_t = _gather_kernel(type_indices.astype(jnp.int32),
                                    type_embedding.reshape(-1))
    out_t = lax.dynamic_update_slice(main_t, tail_t,
                                     (0, NFULL * CHUNK + TAIL_ALIGNED))
    return out_t.T


# speedup vs baseline: 1.5992x; 1.2485x over previous
import jax
import jax.numpy as jnp
from jax import lax
from jax.experimental import pallas as pl
from jax.experimental.pallas import tpu as pltpu
from jax.experimental.pallas import tpu_sc as plsc

NUM_NODES = 100000
NUM_TYPES_ROWS = 64
EMBED_DIM = 64
LANES = 16
CHUNK = 512
NGRP = CHUNK // LANES
NFULL = NUM_NODES // CHUNK
TAIL = NUM_NODES - NFULL * CHUNK
TAIL_ALIGNED = 128
TAIL_REST = TAIL - TAIL_ALIGNED
NUM_WORKERS = 32
KSTEPS = -(-NFULL // NUM_WORKERS)
TAIL_WID = 3

_mesh = plsc.VectorSubcoreMesh(core_axis_name="c", subcore_axis_name="s")


@pl.kernel(
    mesh=_mesh,
    compiler_params=pltpu.CompilerParams(needs_layout_passes=False),
    out_type=(jax.ShapeDtypeStruct((EMBED_DIM, NUM_NODES), jnp.float32),
              jax.ShapeDtypeStruct((EMBED_DIM, TAIL_REST), jnp.float32)),
    scratch_types=[
        pltpu.VMEM((NUM_TYPES_ROWS * EMBED_DIM,), jnp.float32),
        pltpu.VMEM((CHUNK,), jnp.int32),
        pltpu.VMEM((CHUNK,), jnp.int32),
        pltpu.VMEM((EMBED_DIM, CHUNK), jnp.float32),
        pltpu.VMEM((EMBED_DIM, CHUNK), jnp.float32),
        pltpu.VMEM((EMBED_DIM, TAIL_REST), jnp.float32),
        pltpu.SemaphoreType.DMA,
        pltpu.SemaphoreType.DMA,
        pltpu.SemaphoreType.DMA,
        pltpu.SemaphoreType.DMA,
    ],
)
def _gather_kernel(idx_hbm, table_hbm, out_hbm, tail_hbm, tab_v, idx0, idx1,
                   obuf0, obuf1, ptail_v, semi0, semi1, semw0, semw1):
    wid = lax.axis_index("s") * 2 + lax.axis_index("c")
    idx_b = (idx0, idx1)
    obuf_b = (obuf0, obuf1)
    semi_b = (semi0, semi1)
    semw_b = (semw0, semw1)

    pltpu.sync_copy(table_hbm, tab_v)

    def stage_idx(k):
        chunk = k * NUM_WORKERS + wid

        @pl.when(chunk < NFULL)
        def _():
            b = k % 2
            pltpu.async_copy(idx_hbm.at[pl.ds(chunk * CHUNK, CHUNK)],
                             idx_b[b], semi_b[b])

    def compute(k):
        chunk = k * NUM_WORKERS + wid

        @pl.when(chunk < NFULL)
        def _():
            b = k % 2
            pltpu.make_async_copy(idx_hbm.at[pl.ds(chunk * CHUNK, CHUNK)],
                                  idx_b[b], semi_b[b]).wait()
            if k >= 2:
                prev = (k - 2) * NUM_WORKERS + wid
                pltpu.make_async_copy(
                    obuf_b[b],
                    out_hbm.at[:, pl.ds(prev * CHUNK, CHUNK)],
                    semw_b[b]).wait()

            @plsc.parallel_loop(0, NGRP, unroll=2)
            def grp(g):
                iv = idx_b[b][pl.ds(g * LANES, LANES)] * EMBED_DIM
                pend = []
                for c in range(EMBED_DIM):
                    pend.append((c, plsc.load_gather(tab_v, [iv + c])))
                    if len(pend) >= 6:
                        cc, vv = pend.pop(0)
                        obuf_b[b][cc, pl.ds(g * LANES, LANES)] = vv
                for cc, vv in pend:
                    obuf_b[b][cc, pl.ds(g * LANES, LANES)] = vv
            pltpu.async_copy(obuf_b[b],
                             out_hbm.at[:, pl.ds(chunk * CHUNK, CHUNK)],
                             semw_b[b])

    def drain(k):
        chunk = k * NUM_WORKERS + wid

        @pl.when(chunk < NFULL)
        def _():
            b = k % 2
            pltpu.make_async_copy(
                obuf_b[b],
                out_hbm.at[:, pl.ds(chunk * CHUNK, CHUNK)],
                semw_b[b]).wait()

    stage_idx(0)
    for k in range(KSTEPS):
        if k + 1 < KSTEPS:
            stage_idx(k + 1)
        compute(k)
    for k in range(max(KSTEPS - 2, 0), KSTEPS):
        drain(k)

    @pl.when(wid == TAIL_WID)
    def _():
        base = NFULL * CHUNK
        pltpu.sync_copy(idx_hbm.at[pl.ds(base, TAIL)],
                        idx0.at[pl.ds(0, TAIL)])

        @plsc.parallel_loop(0, TAIL_ALIGNED // LANES, unroll=2)
        def grp(g):
            iv = idx0[pl.ds(g * LANES, LANES)] * EMBED_DIM
            for c in range(EMBED_DIM):
                vals = plsc.load_gather(tab_v, [iv + c])
                obuf0[c, pl.ds(g * LANES, LANES)] = vals
        pltpu.sync_copy(obuf0.at[:, pl.ds(0, TAIL_ALIGNED)],
                        out_hbm.at[:, pl.ds(base, TAIL_ALIGNED)])

        @plsc.parallel_loop(0, TAIL_REST // LANES, unroll=1)
        def grp_rest(g):
            iv = idx0[pl.ds(TAIL_ALIGNED + g * LANES, LANES)] * EMBED_DIM
            for c in range(EMBED_DIM):
                vals = plsc.load_gather(tab_v, [iv + c])
                ptail_v[c, pl.ds(g * LANES, LANES)] = vals
        pltpu.sync_copy(ptail_v, tail_hbm)


def kernel(type_indices, type_embedding):
    main_t, tail_t = _gather_kernel(type_indices.astype(jnp.int32),
                                    type_embedding.reshape(-1))
    out_t = lax.dynamic_update_slice(main_t, tail_t,
                                     (0, NFULL * CHUNK + TAIL_ALIGNED))
    return out_t.T


# speedup vs baseline: 3.6130x; 2.2592x over previous
import jax
import jax.numpy as jnp
from jax import lax
from jax.experimental import pallas as pl
from jax.experimental.pallas import tpu as pltpu
from jax.experimental.pallas import tpu_sc as plsc

NUM_NODES = 100000
NUM_TYPES_ROWS = 64
EMBED_DIM = 64
LANES = 16
CHUNK = 512
NGRP = CHUNK // LANES
NFULL = NUM_NODES // CHUNK
TAIL = NUM_NODES - NFULL * CHUNK
TAIL_ALIGNED = 128
TAIL_REST = TAIL - TAIL_ALIGNED
NUM_WORKERS = 32
KSTEPS = -(-NFULL // NUM_WORKERS)
TAIL_WID = 3

_mesh = plsc.VectorSubcoreMesh(core_axis_name="c", subcore_axis_name="s")


@pl.kernel(
    mesh=_mesh,
    compiler_params=pltpu.CompilerParams(needs_layout_passes=False),
    out_type=(jax.ShapeDtypeStruct((EMBED_DIM, NUM_NODES), jnp.float32),
              jax.ShapeDtypeStruct((EMBED_DIM, TAIL_REST), jnp.float32)),
    scratch_types=[
        pltpu.VMEM((NUM_TYPES_ROWS * EMBED_DIM,), jnp.float32),
        pltpu.VMEM((CHUNK,), jnp.int32),
        pltpu.VMEM((CHUNK,), jnp.int32),
        pltpu.VMEM((EMBED_DIM, CHUNK), jnp.float32),
        pltpu.VMEM((EMBED_DIM, CHUNK), jnp.float32),
        pltpu.VMEM((EMBED_DIM, TAIL_REST), jnp.float32),
        pltpu.SemaphoreType.DMA,
        pltpu.SemaphoreType.DMA,
        pltpu.SemaphoreType.DMA,
        pltpu.SemaphoreType.DMA,
    ],
)
def _gather_kernel(idx_hbm, table_hbm, out_hbm, tail_hbm, tab_v, idx0, idx1,
                   obuf0, obuf1, ptail_v, semi0, semi1, semw0, semw1):
    wid = lax.axis_index("s") * 2 + lax.axis_index("c")
    idx_b = (idx0, idx1)
    obuf_b = (obuf0, obuf1)
    semi_b = (semi0, semi1)
    semw_b = (semw0, semw1)

    pltpu.sync_copy(table_hbm, tab_v)

    def stage_idx(k):
        chunk = k * NUM_WORKERS + wid

        @pl.when(chunk < NFULL)
        def _():
            b = k % 2
            pltpu.async_copy(idx_hbm.at[pl.ds(chunk * CHUNK, CHUNK)],
                             idx_b[b], semi_b[b])

    def compute(k):
        chunk = k * NUM_WORKERS + wid

        @pl.when(chunk < NFULL)
        def _():
            b = k % 2
            pltpu.make_async_copy(idx_hbm.at[pl.ds(chunk * CHUNK, CHUNK)],
                                  idx_b[b], semi_b[b]).wait()
            if k >= 2:
                prev = (k - 2) * NUM_WORKERS + wid
                pltpu.make_async_copy(
                    obuf_b[b],
                    out_hbm.at[:, pl.ds(prev * CHUNK, CHUNK)],
                    semw_b[b]).wait()

            @plsc.parallel_loop(0, NGRP, unroll=2)
            def grp(g):
                iv = idx_b[b][pl.ds(g * LANES, LANES)]
                pend = []
                for c in range(EMBED_DIM):
                    pend.append(
                        (c, plsc.load_gather(tab_v, [iv + c * NUM_TYPES_ROWS])))
                    if len(pend) >= 6:
                        cc, vv = pend.pop(0)
                        obuf_b[b][cc, pl.ds(g * LANES, LANES)] = vv
                for cc, vv in pend:
                    obuf_b[b][cc, pl.ds(g * LANES, LANES)] = vv
            pltpu.async_copy(obuf_b[b],
                             out_hbm.at[:, pl.ds(chunk * CHUNK, CHUNK)],
                             semw_b[b])

    def drain(k):
        chunk = k * NUM_WORKERS + wid

        @pl.when(chunk < NFULL)
        def _():
            b = k % 2
            pltpu.make_async_copy(
                obuf_b[b],
                out_hbm.at[:, pl.ds(chunk * CHUNK, CHUNK)],
                semw_b[b]).wait()

    stage_idx(0)
    for k in range(KSTEPS):
        if k + 1 < KSTEPS:
            stage_idx(k + 1)
        compute(k)
    for k in range(max(KSTEPS - 2, 0), KSTEPS):
        drain(k)

    @pl.when(wid == TAIL_WID)
    def _():
        base = NFULL * CHUNK
        pltpu.sync_copy(idx_hbm.at[pl.ds(base, TAIL)],
                        idx0.at[pl.ds(0, TAIL)])

        @plsc.parallel_loop(0, TAIL_ALIGNED // LANES, unroll=2)
        def grp(g):
            iv = idx0[pl.ds(g * LANES, LANES)]
            for c in range(EMBED_DIM):
                vals = plsc.load_gather(tab_v, [iv + c * NUM_TYPES_ROWS])
                obuf0[c, pl.ds(g * LANES, LANES)] = vals
        pltpu.sync_copy(obuf0.at[:, pl.ds(0, TAIL_ALIGNED)],
                        out_hbm.at[:, pl.ds(base, TAIL_ALIGNED)])

        @plsc.parallel_loop(0, TAIL_REST // LANES, unroll=1)
        def grp_rest(g):
            iv = idx0[pl.ds(TAIL_ALIGNED + g * LANES, LANES)]
            for c in range(EMBED_DIM):
                vals = plsc.load_gather(tab_v, [iv + c * NUM_TYPES_ROWS])
                ptail_v[c, pl.ds(g * LANES, LANES)] = vals
        pltpu.sync_copy(ptail_v, tail_hbm)


def kernel(type_indices, type_embedding):
    main_t, tail_t = _gather_kernel(type_indices.astype(jnp.int32),
                                    type_embedding.T.reshape(-1))
    out_t = lax.dynamic_update_slice(main_t, tail_t,
                                     (0, NFULL * CHUNK + TAIL_ALIGNED))
    return out_t.T


# speedup vs baseline: 4.1357x; 1.1447x over previous
import jax
import jax.numpy as jnp
from jax import lax
from jax.experimental import pallas as pl
from jax.experimental.pallas import tpu as pltpu
from jax.experimental.pallas import tpu_sc as plsc

NUM_NODES = 100000
NUM_TYPES_ROWS = 64
EMBED_DIM = 64
LANES = 16
CHUNK = 512
NGRP = CHUNK // LANES
NFULL = NUM_NODES // CHUNK
TAIL = NUM_NODES - NFULL * CHUNK
TAIL_ALIGNED = 128
TAIL_REST = TAIL - TAIL_ALIGNED
NUM_WORKERS = 32
KSTEPS = -(-NFULL // NUM_WORKERS)
TAIL_WID = 3

_mesh = plsc.VectorSubcoreMesh(core_axis_name="c", subcore_axis_name="s")


@pl.kernel(
    mesh=_mesh,
    compiler_params=pltpu.CompilerParams(needs_layout_passes=False),
    out_type=(jax.ShapeDtypeStruct((EMBED_DIM, NUM_NODES), jnp.float32),
              jax.ShapeDtypeStruct((EMBED_DIM, TAIL_REST), jnp.float32)),
    scratch_types=[
        pltpu.VMEM((NUM_TYPES_ROWS * EMBED_DIM,), jnp.float32),
        pltpu.VMEM((CHUNK,), jnp.int32),
        pltpu.VMEM((CHUNK,), jnp.int32),
        pltpu.VMEM((EMBED_DIM, CHUNK), jnp.float32),
        pltpu.VMEM((EMBED_DIM, CHUNK), jnp.float32),
        pltpu.VMEM((EMBED_DIM, TAIL_REST), jnp.float32),
        pltpu.SemaphoreType.DMA,
        pltpu.SemaphoreType.DMA,
        pltpu.SemaphoreType.DMA,
        pltpu.SemaphoreType.DMA,
    ],
)
def _gather_kernel(idx_hbm, table_hbm, out_hbm, tail_hbm, tab_v, idx0, idx1,
                   obuf0, obuf1, ptail_v, semi0, semi1, semw0, semw1):
    wid = lax.axis_index("s") * 2 + lax.axis_index("c")
    idx_b = (idx0, idx1)
    obuf_b = (obuf0, obuf1)
    semi_b = (semi0, semi1)
    semw_b = (semw0, semw1)

    pltpu.sync_copy(table_hbm, tab_v)

    def stage_idx(k):
        chunk = k * NUM_WORKERS + wid

        @pl.when(chunk < NFULL)
        def _():
            b = k % 2
            pltpu.async_copy(idx_hbm.at[pl.ds(chunk * CHUNK, CHUNK)],
                             idx_b[b], semi_b[b])

    def compute(k):
        chunk = k * NUM_WORKERS + wid

        @pl.when(chunk < NFULL)
        def _():
            b = k % 2
            pltpu.make_async_copy(idx_hbm.at[pl.ds(chunk * CHUNK, CHUNK)],
                                  idx_b[b], semi_b[b]).wait()
            if k >= 2:
                prev = (k - 2) * NUM_WORKERS + wid
                pltpu.make_async_copy(
                    obuf_b[b],
                    out_hbm.at[:, pl.ds(prev * CHUNK, CHUNK)],
                    semw_b[b]).wait()

            @plsc.parallel_loop(0, NGRP, unroll=4)
            def grp(g):
                iv = idx_b[b][pl.ds(g * LANES, LANES)]
                pend = []
                for c in range(EMBED_DIM):
                    pend.append(
                        (c, plsc.load_gather(tab_v, [iv + c * NUM_TYPES_ROWS])))
                    if len(pend) >= 8:
                        cc, vv = pend.pop(0)
                        obuf_b[b][cc, pl.ds(g * LANES, LANES)] = vv
                for cc, vv in pend:
                    obuf_b[b][cc, pl.ds(g * LANES, LANES)] = vv
            pltpu.async_copy(obuf_b[b],
                             out_hbm.at[:, pl.ds(chunk * CHUNK, CHUNK)],
                             semw_b[b])

    def drain(k):
        chunk = k * NUM_WORKERS + wid

        @pl.when(chunk < NFULL)
        def _():
            b = k % 2
            pltpu.make_async_copy(
                obuf_b[b],
                out_hbm.at[:, pl.ds(chunk * CHUNK, CHUNK)],
                semw_b[b]).wait()

    stage_idx(0)
    for k in range(KSTEPS):
        if k + 1 < KSTEPS:
            stage_idx(k + 1)
        compute(k)
    for k in range(max(KSTEPS - 2, 0), KSTEPS):
        drain(k)

    @pl.when(wid == TAIL_WID)
    def _():
        base = NFULL * CHUNK
        pltpu.sync_copy(idx_hbm.at[pl.ds(base, TAIL)],
                        idx0.at[pl.ds(0, TAIL)])

        @plsc.parallel_loop(0, TAIL_ALIGNED // LANES, unroll=2)
        def grp(g):
            iv = idx0[pl.ds(g * LANES, LANES)]
            for c in range(EMBED_DIM):
                vals = plsc.load_gather(tab_v, [iv + c * NUM_TYPES_ROWS])
                obuf0[c, pl.ds(g * LANES, LANES)] = vals
        pltpu.sync_copy(obuf0.at[:, pl.ds(0, TAIL_ALIGNED)],
                        out_hbm.at[:, pl.ds(base, TAIL_ALIGNED)])

        @plsc.parallel_loop(0, TAIL_REST // LANES, unroll=1)
        def grp_rest(g):
            iv = idx0[pl.ds(TAIL_ALIGNED + g * LANES, LANES)]
            for c in range(EMBED_DIM):
                vals = plsc.load_gather(tab_v, [iv + c * NUM_TYPES_ROWS])
                ptail_v[c, pl.ds(g * LANES, LANES)] = vals
        pltpu.sync_copy(ptail_v, tail_hbm)


def kernel(type_indices, type_embedding):
    main_t, tail_t = _gather_kernel(type_indices.astype(jnp.int32),
                                    type_embedding.T.reshape(-1))
    out_t = lax.dynamic_update_slice(main_t, tail_t,
                                     (0, NFULL * CHUNK + TAIL_ALIGNED))
    return out_t.T
